# Initial kernel scaffold; baseline (speedup 1.0000x reference)
import functools

import jax
import jax.numpy as jnp
from jax import lax
from jax.experimental import pallas as pl
from jax.experimental.pallas import tpu as pltpu
from jax.experimental.pallas import tpu_sc as plsc

N = 10000
E = 320000
CH = 128
NC, NS = 2, 16
NW = NC * NS
NP = 10240
RPT = NP // NS
CHUNK = 128
CPW = 80
EPW = CPW * CHUNK
EPAD = NW * EPW
HW = 16

_MESH = plsc.VectorSubcoreMesh(core_axis_name="c", subcore_axis_name="s")


@functools.partial(
    pl.kernel,
    out_type=jax.ShapeDtypeStruct((NC, NP, HW), jnp.float32),
    mesh=_MESH,
    scratch_types=[
        pltpu.VMEM((CPW, CHUNK), jnp.int32),
        pltpu.VMEM((CHUNK, HW), jnp.float32),
        pltpu.VMEM((RPT, HW), jnp.float32),
        pltpu.VMEM_SHARED((NP, HW), jnp.float32),
    ],
)
def _deg_kernel(dst_hbm, out_hbm, idx_v, ones_v, zeros_v, hist_s):
    c = lax.axis_index("c")
    s = lax.axis_index("s")
    wid = c * NS + s
    lane = lax.iota(jnp.int32, 16)
    e0 = jnp.where(lane == 0, 1.0, 0.0).astype(jnp.float32)
    z16 = jnp.zeros((16,), jnp.float32)

    def fill_zeros(i, carry):
        zeros_v[i, :] = z16
        return carry

    lax.fori_loop(0, RPT, fill_zeros, 0)

    def fill_ones(i, carry):
        ones_v[i, :] = e0
        return carry

    lax.fori_loop(0, CHUNK, fill_ones, 0)

    pltpu.sync_copy(zeros_v, hist_s.at[pl.ds(s * RPT, RPT)])
    plsc.subcore_barrier()

    pltpu.sync_copy(dst_hbm.at[wid], idx_v)

    def add_chunk(j, carry):
        pltpu.sync_copy(ones_v, hist_s.at[idx_v.at[j]], add=True)
        return carry

    lax.fori_loop(0, CPW, add_chunk, 0)
    plsc.subcore_barrier()
    pltpu.sync_copy(hist_s.at[pl.ds(s * RPT, RPT)],
                    out_hbm.at[c, pl.ds(s * RPT, RPT)])


@functools.partial(
    pl.kernel,
    out_type=jax.ShapeDtypeStruct((NC, NP, CH), jnp.float32),
    mesh=_MESH,
    scratch_types=[
        pltpu.VMEM((CPW, CHUNK), jnp.int32),
        pltpu.VMEM((CPW, CHUNK), jnp.int32),
        pltpu.VMEM((CHUNK, CH), jnp.float32),
        pltpu.VMEM((CHUNK, CH), jnp.float32),
        pltpu.VMEM_SHARED((NP, CH), jnp.float32),
        pltpu.SemaphoreType.DMA,
    ],
)
def _agg_kernel(h2_hbm, src_hbm, dst_hbm, out_hbm,
                srcv, dstv, rowbuf, zbuf, acc_s, sem):
    c = lax.axis_index("c")
    s = lax.axis_index("s")
    wid = c * NS + s
    z16 = jnp.zeros((16,), jnp.float32)

    def fill_zeros(i, carry):
        for k in range(CH // 16):
            zbuf[i, pl.ds(k * 16, 16)] = z16
        return carry

    lax.fori_loop(0, CHUNK, fill_zeros, 0)
    for k in range(RPT // CHUNK):
        pltpu.sync_copy(zbuf, acc_s.at[pl.ds(s * RPT + k * CHUNK, CHUNK)])
    plsc.subcore_barrier()

    pltpu.sync_copy(src_hbm.at[wid], srcv)
    pltpu.sync_copy(dst_hbm.at[wid], dstv)

    def chunk(j, carry):
        pltpu.async_copy(h2_hbm.at[srcv.at[j]], rowbuf, sem).wait()
        pltpu.sync_copy(rowbuf, acc_s.at[dstv.at[j]], add=True)
        return carry

    lax.fori_loop(0, CPW, chunk, 0)
    plsc.subcore_barrier()
    pltpu.sync_copy(acc_s.at[pl.ds(s * RPT, RPT)],
                    out_hbm.at[c, pl.ds(s * RPT, RPT)])


BM = 1024


def _h2_body(hist_ref, x_ref, w_ref, h2_ref):
    deg = hist_ref[0, :, 0] + hist_ref[1, :, 0] + 1.0
    dinv = lax.rsqrt(deg)
    h = jnp.dot(x_ref[...], w_ref[...], preferred_element_type=jnp.float32)
    h2_ref[...] = h * dinv[:, None]


def _h2_call(hist, xp, W):
    return pl.pallas_call(
        _h2_body,
        grid=(NP // BM,),
        in_specs=[
            pl.BlockSpec((NC, BM, HW), lambda i: (0, i, 0)),
            pl.BlockSpec((BM, CH), lambda i: (i, 0)),
            pl.BlockSpec((CH, CH), lambda i: (0, 0)),
        ],
        out_specs=pl.BlockSpec((BM, CH), lambda i: (i, 0)),
        out_shape=jax.ShapeDtypeStruct((NP, CH), jnp.float32),
    )(hist, xp, W)


def _fin_body(hist_ref, parts_ref, h2_ref, b_ref, o_ref):
    deg = hist_ref[0, :, 0] + hist_ref[1, :, 0] + 1.0
    dinv = lax.rsqrt(deg)
    agg = parts_ref[0] + parts_ref[1] + h2_ref[...]
    o_ref[...] = jnp.maximum(agg * dinv[:, None] + b_ref[...], 0.0)


def _fin_call(hist, parts, h2, b2):
    return pl.pallas_call(
        _fin_body,
        grid=(NP // BM,),
        in_specs=[
            pl.BlockSpec((NC, BM, HW), lambda i: (0, i, 0)),
            pl.BlockSpec((NC, BM, CH), lambda i: (0, i, 0)),
            pl.BlockSpec((BM, CH), lambda i: (i, 0)),
            pl.BlockSpec((1, CH), lambda i: (0, 0)),
        ],
        out_specs=pl.BlockSpec((BM, CH), lambda i: (i, 0)),
        out_shape=jax.ShapeDtypeStruct((NP, CH), jnp.float32),
    )(hist, parts, h2, b2)


def kernel(x, edge_index, W, b):
    src = edge_index[0].astype(jnp.int32)
    dst = edge_index[1].astype(jnp.int32)
    pad = EPAD - E
    src_p = jnp.concatenate([src, jnp.zeros((pad,), jnp.int32)])
    dst_p = jnp.concatenate([dst, jnp.full((pad,), N, jnp.int32)])
    srcr = src_p.reshape(NW, CPW, CHUNK)
    dstr = dst_p.reshape(NW, CPW, CHUNK)
    xp = jnp.concatenate([x, jnp.zeros((NP - N, CH), x.dtype)])

    hist = _deg_kernel(dstr)
    h2 = _h2_call(hist, xp, W)
    parts = _agg_kernel(h2, srcr, dstr)
    out = _fin_call(hist, parts, h2, b.reshape(1, CH))
    return out[:N]


# speedup vs baseline: 11.5163x; 11.5163x over previous
import functools

import jax
import jax.numpy as jnp
from jax import lax
from jax.experimental import pallas as pl
from jax.experimental.pallas import tpu as pltpu
from jax.experimental.pallas import tpu_sc as plsc

N = 10000
E = 320000
CH = 128
NC, NS = 2, 16
NW = NC * NS
NP = 10240
RPT = NP // NS
CHUNK = 128
KPT = RPT // CHUNK
CPW = 80
EPW = CPW * CHUNK
EPAD = NW * EPW
HW = 128

_MESH = plsc.VectorSubcoreMesh(core_axis_name="c", subcore_axis_name="s")


def _fill_row_indices(idxbuf, base, lane):
    for k in range(KPT):
        for j in range(CHUNK // 16):
            idxbuf[k, pl.ds(j * 16, 16)] = base + (k * CHUNK + j * 16) + lane


@functools.partial(
    pl.kernel,
    out_type=jax.ShapeDtypeStruct((NC * NP, HW), jnp.float32),
    mesh=_MESH,
    scratch_types=[
        pltpu.VMEM((CPW, CHUNK), jnp.int32),
        pltpu.VMEM((CHUNK, HW), jnp.float32),
        pltpu.VMEM((CHUNK, HW), jnp.float32),
        pltpu.VMEM((KPT, CHUNK), jnp.int32),
        pltpu.VMEM_SHARED((NP, HW), jnp.float32),
    ],
)
def _deg_kernel(dst_hbm, out_hbm, idx_v, ones_v, buf_v, idxbuf, hist_s):
    c = lax.axis_index("c")
    s = lax.axis_index("s")
    wid = c * NS + s
    lane = lax.iota(jnp.int32, 16)
    e0 = jnp.where(lane == 0, 1.0, 0.0).astype(jnp.float32)
    z16 = jnp.zeros((16,), jnp.float32)

    def fill_rows(i, carry):
        ones_v[i, pl.ds(0, 16)] = e0
        buf_v[i, pl.ds(0, 16)] = z16
        for k in range(1, HW // 16):
            ones_v[i, pl.ds(k * 16, 16)] = z16
            buf_v[i, pl.ds(k * 16, 16)] = z16
        return carry

    lax.fori_loop(0, CHUNK, fill_rows, 0)
    _fill_row_indices(idxbuf, s * RPT, lane)

    for k in range(KPT):
        pltpu.sync_copy(buf_v, hist_s.at[idxbuf.at[k]])
    plsc.subcore_barrier()

    pltpu.sync_copy(dst_hbm.at[wid], idx_v)

    def add_chunk(j, carry):
        pltpu.sync_copy(ones_v, hist_s.at[idx_v.at[j]], add=True)
        return carry

    lax.fori_loop(0, CPW, add_chunk, 0)
    plsc.subcore_barrier()

    flatbase = c * NP + s * RPT
    for k in range(KPT):
        pltpu.sync_copy(hist_s.at[idxbuf.at[k]], buf_v)
        pltpu.sync_copy(buf_v, out_hbm.at[pl.ds(flatbase + k * CHUNK, CHUNK)])


@functools.partial(
    pl.kernel,
    out_type=jax.ShapeDtypeStruct((NC * NP, CH), jnp.float32),
    mesh=_MESH,
    scratch_types=[
        pltpu.VMEM((CPW, CHUNK), jnp.int32),
        pltpu.VMEM((CPW, CHUNK), jnp.int32),
        pltpu.VMEM((CHUNK, CH), jnp.float32),
        pltpu.VMEM((KPT, CHUNK), jnp.int32),
        pltpu.VMEM_SHARED((NP, CH), jnp.float32),
        pltpu.SemaphoreType.DMA,
    ],
)
def _agg_kernel(h2_hbm, src_hbm, dst_hbm, out_hbm,
                srcv, dstv, rowbuf, idxbuf, acc_s, sem):
    c = lax.axis_index("c")
    s = lax.axis_index("s")
    wid = c * NS + s
    lane = lax.iota(jnp.int32, 16)
    z16 = jnp.zeros((16,), jnp.float32)

    def fill_zeros(i, carry):
        for k in range(CH // 16):
            rowbuf[i, pl.ds(k * 16, 16)] = z16
        return carry

    lax.fori_loop(0, CHUNK, fill_zeros, 0)
    _fill_row_indices(idxbuf, s * RPT, lane)

    for k in range(KPT):
        pltpu.sync_copy(rowbuf, acc_s.at[idxbuf.at[k]])
    plsc.subcore_barrier()

    pltpu.sync_copy(src_hbm.at[wid], srcv)
    pltpu.sync_copy(dst_hbm.at[wid], dstv)

    def chunk(j, carry):
        pltpu.async_copy(h2_hbm.at[srcv.at[j]], rowbuf, sem).wait()
        pltpu.sync_copy(rowbuf, acc_s.at[dstv.at[j]], add=True)
        return carry

    lax.fori_loop(0, CPW, chunk, 0)
    plsc.subcore_barrier()

    flatbase = c * NP + s * RPT
    for k in range(KPT):
        pltpu.sync_copy(acc_s.at[idxbuf.at[k]], rowbuf)
        pltpu.sync_copy(rowbuf, out_hbm.at[pl.ds(flatbase + k * CHUNK, CHUNK)])


BM = 1024


def _h2_body(hist_ref, x_ref, w_ref, h2_ref):
    deg = hist_ref[0, :, 0] + hist_ref[1, :, 0] + 1.0
    dinv = lax.rsqrt(deg)
    h = jnp.dot(x_ref[...], w_ref[...], preferred_element_type=jnp.float32)
    h2_ref[...] = h * dinv[:, None]


def _h2_call(hist, xp, W):
    return pl.pallas_call(
        _h2_body,
        grid=(NP // BM,),
        in_specs=[
            pl.BlockSpec((NC, BM, HW), lambda i: (0, i, 0)),
            pl.BlockSpec((BM, CH), lambda i: (i, 0)),
            pl.BlockSpec((CH, CH), lambda i: (0, 0)),
        ],
        out_specs=pl.BlockSpec((BM, CH), lambda i: (i, 0)),
        out_shape=jax.ShapeDtypeStruct((NP, CH), jnp.float32),
    )(hist, xp, W)


def _fin_body(hist_ref, parts_ref, h2_ref, b_ref, o_ref):
    deg = hist_ref[0, :, 0] + hist_ref[1, :, 0] + 1.0
    dinv = lax.rsqrt(deg)
    agg = parts_ref[0] + parts_ref[1] + h2_ref[...]
    o_ref[...] = jnp.maximum(agg * dinv[:, None] + b_ref[...], 0.0)


def _fin_call(hist, parts, h2, b2):
    return pl.pallas_call(
        _fin_body,
        grid=(NP // BM,),
        in_specs=[
            pl.BlockSpec((NC, BM, HW), lambda i: (0, i, 0)),
            pl.BlockSpec((NC, BM, CH), lambda i: (0, i, 0)),
            pl.BlockSpec((BM, CH), lambda i: (i, 0)),
            pl.BlockSpec((1, CH), lambda i: (0, 0)),
        ],
        out_specs=pl.BlockSpec((BM, CH), lambda i: (i, 0)),
        out_shape=jax.ShapeDtypeStruct((NP, CH), jnp.float32),
    )(hist, parts, h2, b2)


def kernel(x, edge_index, W, b):
    src = edge_index[0].astype(jnp.int32)
    dst = edge_index[1].astype(jnp.int32)
    pad = EPAD - E
    src_p = jnp.concatenate([src, jnp.zeros((pad,), jnp.int32)])
    dst_p = jnp.concatenate([dst, jnp.full((pad,), N, jnp.int32)])
    srcr = src_p.reshape(NW, CPW, CHUNK)
    dstr = dst_p.reshape(NW, CPW, CHUNK)
    xp = jnp.concatenate([x, jnp.zeros((NP - N, CH), x.dtype)])

    hist = _deg_kernel(dstr).reshape(NC, NP, HW)
    h2 = _h2_call(hist, xp, W)
    parts = _agg_kernel(h2, srcr, dstr).reshape(NC, NP, CH)
    out = _fin_call(hist, parts, h2, b.reshape(1, CH))
    return out[:N]
